# baseline (speedup 1.0000x reference)
import jax
import jax.numpy as jnp
from jax import lax
from jax.experimental import pallas as pl
from jax.experimental.pallas import tpu as pltpu
from jax.experimental.pallas import tpu_sc as plsc

B, S, H = 64, 512, 768
M = 1024
CS = 96
NSUB = 8
TP = 128
NROW = B * S * NSUB
PERIOD = S * NSUB
NC, NS = 2, 16
NW = NC * NS
WS = PERIOD // NW
ORS = S // NW
NCHUNK = NROW // PERIOD
L = 16


def _sc_body(xf_hbm, yf_hbm, xv_hbm, yv_hbm, pe_hbm, outv_hbm,
             xi_v, yi_v, pe_v, ax_v, ay_v, ov_v, gsem, wsem):
    wid = lax.axis_index("s") * NC + lax.axis_index("c")
    pltpu.sync_copy(pe_hbm.at[pl.ds(wid * ORS, ORS)], pe_v)
    offs = (lax.iota(jnp.int32, L) % NSUB) * M

    def load_idx(p, par):
        base = p * PERIOD + wid * WS
        pltpu.sync_copy(xf_hbm.at[pl.ds(base, WS)], xi_v.at[par])
        pltpu.sync_copy(yf_hbm.at[pl.ds(base, WS)], yi_v.at[par])
        for g in range(WS // L):
            sl = pl.ds(g * L, L)
            xi_v[par, sl] = xi_v[par, sl] + offs
            yi_v[par, sl] = yi_v[par, sl] + offs

    def issue(par):
        pltpu.async_copy(xv_hbm.at[xi_v.at[par]], ax_v.at[par], gsem)
        pltpu.async_copy(yv_hbm.at[yi_v.at[par]], ay_v.at[par], gsem)

    def wait_gathers(par):
        pltpu.make_async_copy(xv_hbm.at[xi_v.at[par]], ax_v.at[par],
                              gsem).wait()
        pltpu.make_async_copy(xv_hbm.at[xi_v.at[par]], ax_v.at[par],
                              gsem).wait()

    def compute(par):
        @plsc.parallel_loop(0, ORS, 1, unroll=4)
        def row(r):
            for i in range(NSUB):
                for c in range(CS // L):
                    dst = pl.ds(i * CS + c * L, L)
                    src = pl.ds(c * L, L)
                    ov_v[par, r, dst] = (ax_v[par, r * NSUB + i, src]
                                         + ay_v[par, r * NSUB + i, src]
                                         + pe_v[r, dst])

    load_idx(0, 0)
    issue(0)

    def giter(g, carry):
        for q in (0, 1):
            p = 2 * g + q
            par = q
            npar = 1 - q
            mbase = p * S + wid * ORS

            if q == 0:
                load_idx(p + 1, npar)
                issue(npar)
            else:
                @pl.when(g < NCHUNK // 2 - 1)
                def _():
                    load_idx(p + 1, npar)
                    issue(npar)

            @pl.when(p >= 2)
            def _():
                pltpu.make_async_copy(ov_v.at[par],
                                      outv_hbm.at[pl.ds(mbase, ORS)],
                                      wsem).wait()

            wait_gathers(par)
            compute(par)
            pltpu.async_copy(ov_v.at[par], outv_hbm.at[pl.ds(mbase, ORS)],
                             wsem)
        return carry

    lax.fori_loop(0, NCHUNK // 2, giter, 0)
    pltpu.make_async_copy(ov_v.at[0], outv_hbm.at[pl.ds(0, ORS)], wsem).wait()
    pltpu.make_async_copy(ov_v.at[1], outv_hbm.at[pl.ds(0, ORS)], wsem).wait()


def _tc_body(xf_ref, yf_ref, xyt_ref, pe_ref, outt_ref):
    iota = lax.broadcasted_iota(jnp.int32, (S, M), 1)
    parts = []
    for i in range(NSUB):
        ox = (xf_ref[0, :, i][:, None] == iota).astype(jnp.bfloat16)
        oy = (yf_ref[0, :, i][:, None] == iota).astype(jnp.bfloat16)
        oh = jnp.concatenate([ox, oy], axis=1)
        parts.append(jnp.dot(oh, xyt_ref[i],
                             preferred_element_type=jnp.float32))
    outt_ref[0] = jnp.concatenate(parts, axis=-1) + pe_ref[...]


def kernel(x_feature, y_feature, x_tables_v, y_tables_v, x_tables_t,
           y_tables_t, pe):
    xf = x_feature.reshape(NROW)
    yf = y_feature.reshape(NROW)

    def flat_pad(t):
        t2 = t.reshape(NSUB * M, CS)
        return jnp.pad(t2, ((0, 0), (0, TP - CS)))

    xv = flat_pad(x_tables_v)
    yv = flat_pad(y_tables_v)
    pe_rows = pe.reshape(S, H)
    mesh = plsc.VectorSubcoreMesh(core_axis_name="c", subcore_axis_name="s",
                                  num_cores=NC, num_subcores=NS)
    outv = pl.kernel(
        _sc_body,
        out_type=jax.ShapeDtypeStruct((B * S, H), jnp.float32),
        mesh=mesh,
        scratch_types=[
            pltpu.VMEM((2, WS), jnp.int32),
            pltpu.VMEM((2, WS), jnp.int32),
            pltpu.VMEM((ORS, H), jnp.float32),
            pltpu.VMEM((2, WS, TP), jnp.float32),
            pltpu.VMEM((2, WS, TP), jnp.float32),
            pltpu.VMEM((2, ORS, H), jnp.float32),
            pltpu.SemaphoreType.DMA,
            pltpu.SemaphoreType.DMA,
        ],
    )(xf, yf, xv, yv, pe_rows)

    xyt = jnp.concatenate([x_tables_t, y_tables_t],
                          axis=1).astype(jnp.bfloat16)
    tab_spec = pl.BlockSpec((NSUB, 2 * M, CS), lambda b: (0, 0, 0))
    outt = pl.pallas_call(
        _tc_body,
        grid=(B,),
        in_specs=[
            pl.BlockSpec((1, S, NSUB), lambda b: (b, 0, 0)),
            pl.BlockSpec((1, S, NSUB), lambda b: (b, 0, 0)),
            tab_spec,
            pl.BlockSpec((S, H), lambda b: (0, 0)),
        ],
        out_specs=pl.BlockSpec((1, S, H), lambda b: (b, 0, 0)),
        out_shape=jax.ShapeDtypeStruct((B, S, H), jnp.float32),
    )(x_feature, y_feature, xyt, pe_rows)

    return outv.reshape(B, S, H), outt


# speedup vs baseline: 1.0276x; 1.0276x over previous
import jax
import jax.numpy as jnp
from jax import lax
from jax.experimental import pallas as pl
from jax.experimental.pallas import tpu as pltpu
from jax.experimental.pallas import tpu_sc as plsc

B, S, H = 64, 512, 768
M = 1024
CS = 96
NSUB = 8
TP = 128
NROW = B * S * NSUB
PERIOD = S * NSUB
NC, NS = 2, 16
NW = NC * NS
WS = PERIOD // NW
ORS = S // NW
NCHUNK = NROW // PERIOD
L = 16


def _sc_body(xf_hbm, yf_hbm, xv_hbm, yv_hbm, pe_hbm, outv_hbm,
             xi_v, yi_v, pe_v, ax_v, ay_v, ov_v, gsem, wsem):
    wid = lax.axis_index("s") * NC + lax.axis_index("c")
    pltpu.sync_copy(pe_hbm.at[pl.ds(wid * ORS, ORS)], pe_v)
    pltpu.sync_copy(xf_hbm.at[:, pl.ds(wid * WS, WS)], xi_v)
    pltpu.sync_copy(yf_hbm.at[:, pl.ds(wid * WS, WS)], yi_v)
    offs = (lax.iota(jnp.int32, L) % NSUB) * M

    @plsc.parallel_loop(0, NCHUNK, 1, unroll=2)
    def _addoffs(p):
        for g in range(WS // L):
            sl = pl.ds(g * L, L)
            xi_v[p, sl] = xi_v[p, sl] + offs
            yi_v[p, sl] = yi_v[p, sl] + offs

    def issue(p, par):
        pltpu.async_copy(xv_hbm.at[xi_v.at[p]], ax_v.at[par], gsem)
        pltpu.async_copy(yv_hbm.at[yi_v.at[p]], ay_v.at[par], gsem)

    def wait_gathers(p, par):
        pltpu.make_async_copy(xv_hbm.at[xi_v.at[p]], ax_v.at[par],
                              gsem).wait()
        pltpu.make_async_copy(xv_hbm.at[xi_v.at[p]], ax_v.at[par],
                              gsem).wait()

    def compute(par):
        @plsc.parallel_loop(0, ORS, 1, unroll=4)
        def row(r):
            for i in range(NSUB):
                for c in range(CS // L):
                    dst = pl.ds(i * CS + c * L, L)
                    src = pl.ds(c * L, L)
                    ov_v[par, r, dst] = (ax_v[par, r * NSUB + i, src]
                                         + ay_v[par, r * NSUB + i, src]
                                         + pe_v[r, dst])

    issue(0, 0)

    def giter(g, carry):
        for q in (0, 1):
            p = 2 * g + q
            par = q
            npar = 1 - q
            mbase = p * S + wid * ORS

            if q == 0:
                issue(p + 1, npar)
            else:
                @pl.when(g < NCHUNK // 2 - 1)
                def _():
                    issue(p + 1, npar)

            @pl.when(p >= 2)
            def _():
                pltpu.make_async_copy(ov_v.at[par],
                                      outv_hbm.at[pl.ds(mbase, ORS)],
                                      wsem).wait()

            wait_gathers(p, par)
            compute(par)
            pltpu.async_copy(ov_v.at[par], outv_hbm.at[pl.ds(mbase, ORS)],
                             wsem)
        return carry

    lax.fori_loop(0, NCHUNK // 2, giter, 0)
    pltpu.make_async_copy(ov_v.at[0], outv_hbm.at[pl.ds(0, ORS)], wsem).wait()
    pltpu.make_async_copy(ov_v.at[1], outv_hbm.at[pl.ds(0, ORS)], wsem).wait()


def _tc_body(xf_ref, yf_ref, xt_ref, yt_ref, pe_ref, outt_ref):
    iota = lax.broadcasted_iota(jnp.int32, (S, M), 1)
    parts = []
    for i in range(NSUB):
        ox = (xf_ref[0, :, i][:, None] == iota).astype(jnp.bfloat16)
        oy = (yf_ref[0, :, i][:, None] == iota).astype(jnp.bfloat16)
        parts.append(jnp.dot(ox, xt_ref[i], preferred_element_type=jnp.float32)
                     + jnp.dot(oy, yt_ref[i],
                               preferred_element_type=jnp.float32))
    outt_ref[0] = jnp.concatenate(parts, axis=-1) + pe_ref[...]


def kernel(x_feature, y_feature, x_tables_v, y_tables_v, x_tables_t,
           y_tables_t, pe):
    xf = x_feature.reshape(NCHUNK, PERIOD)
    yf = y_feature.reshape(NCHUNK, PERIOD)

    def flat_pad(t):
        t2 = t.reshape(NSUB * M, CS)
        return jnp.pad(t2, ((0, 0), (0, TP - CS)))

    xv = flat_pad(x_tables_v)
    yv = flat_pad(y_tables_v)
    pe_rows = pe.reshape(S, H)
    mesh = plsc.VectorSubcoreMesh(core_axis_name="c", subcore_axis_name="s",
                                  num_cores=NC, num_subcores=NS)
    outv = pl.kernel(
        _sc_body,
        out_type=jax.ShapeDtypeStruct((B * S, H), jnp.float32),
        mesh=mesh,
        scratch_types=[
            pltpu.VMEM((NCHUNK, WS), jnp.int32),
            pltpu.VMEM((NCHUNK, WS), jnp.int32),
            pltpu.VMEM((ORS, H), jnp.float32),
            pltpu.VMEM((2, WS, TP), jnp.float32),
            pltpu.VMEM((2, WS, TP), jnp.float32),
            pltpu.VMEM((2, ORS, H), jnp.float32),
            pltpu.SemaphoreType.DMA,
            pltpu.SemaphoreType.DMA,
        ],
    )(xf, yf, xv, yv, pe_rows)

    xt = x_tables_t.astype(jnp.bfloat16)
    yt = y_tables_t.astype(jnp.bfloat16)
    tab_spec = pl.BlockSpec((NSUB, M, CS), lambda b: (0, 0, 0))
    outt = pl.pallas_call(
        _tc_body,
        grid=(B,),
        in_specs=[
            pl.BlockSpec((1, S, NSUB), lambda b: (b, 0, 0)),
            pl.BlockSpec((1, S, NSUB), lambda b: (b, 0, 0)),
            tab_spec, tab_spec,
            pl.BlockSpec((S, H), lambda b: (0, 0)),
        ],
        out_specs=pl.BlockSpec((1, S, H), lambda b: (b, 0, 0)),
        out_shape=jax.ShapeDtypeStruct((B, S, H), jnp.float32),
    )(x_feature, y_feature, xt, yt, pe_rows)

    return outv.reshape(B, S, H), outt
